# baseline (speedup 1.0000x reference)
import functools

import jax
import jax.numpy as jnp
from jax import lax
from jax.experimental import pallas as pl
from jax.experimental.pallas import tpu as pltpu
from jax.experimental.pallas import tpu_sc as plsc

_NF = 256
_NC = 1024
_N = 8 * 24 * 24
_BLK = 512
_NBLK = _N // _BLK

_NW = 32
_BPW = _N // _NW
_CH = 2
_CB = _BPW // _CH


def _tc_body(x_ref, c_ref, idx_ref, ct_ref, loss_ref):
    i = pl.program_id(0)
    x = x_ref[...]
    c = c_ref[...]
    mm = jnp.dot(x, c, preferred_element_type=jnp.float32)
    c_sq = jnp.sum(c * c, axis=0, keepdims=True)
    x_sq = jnp.sum(x * x, axis=1, keepdims=True)
    dist = (c_sq + x_sq) - 2.0 * mm
    m = jnp.min(dist, axis=1, keepdims=True)
    ids = lax.broadcasted_iota(jnp.int32, dist.shape, 1)
    idx = jnp.min(jnp.where(dist == m, ids, _NC), axis=1)
    idx_ref[0, 0, :] = idx

    @pl.when(i == 0)
    def _():
        ct_ref[...] = c.T
        loss_ref[...] = jnp.zeros((1, 1), jnp.float32)

    loss_ref[...] += jnp.sum(m, axis=(0, 1), keepdims=True)


_tc_call = pl.pallas_call(
    _tc_body,
    grid=(_NBLK,),
    in_specs=[
        pl.BlockSpec((_BLK, _NF), lambda i: (i, 0)),
        pl.BlockSpec((_NF, _NC), lambda i: (0, 0)),
    ],
    out_specs=[
        pl.BlockSpec((1, 1, _BLK), lambda i: (i, 0, 0)),
        pl.BlockSpec((_NC, _NF), lambda i: (0, 0)),
        pl.BlockSpec((1, 1), lambda i: (0, 0)),
    ],
    out_shape=[
        jax.ShapeDtypeStruct((_NBLK, 1, _BLK), jnp.int32),
        jax.ShapeDtypeStruct((_NC, _NF), jnp.float32),
        jax.ShapeDtypeStruct((1, 1), jnp.float32),
    ],
)


@functools.cache
def _sc_gather_call():
    @functools.partial(
        pl.kernel,
        mesh=plsc.VectorSubcoreMesh(core_axis_name="c", subcore_axis_name="s"),
        out_type=jax.ShapeDtypeStruct((_N, _NF), jnp.float32),
        scratch_types=[
            pltpu.VMEM((_CH, _CB), jnp.int32),
            pltpu.VMEM((_BPW, _NF), jnp.float32),
            pltpu.SemaphoreType.DMA,
        ],
    )
    def _sc_gather(table_hbm, idx_hbm, out_hbm, idx_v, rows_v, sem):
        wid = lax.axis_index("s") * 2 + lax.axis_index("c")
        pltpu.sync_copy(idx_hbm.at[wid], idx_v)
        cps = [
            pltpu.async_copy(
                table_hbm.at[idx_v.at[j]], rows_v.at[pl.ds(j * _CB, _CB)], sem
            )
            for j in range(_CH)
        ]
        for cp in cps:
            cp.wait()
        pltpu.sync_copy(rows_v, out_hbm.at[pl.ds(wid * _BPW, _BPW)])

    return _sc_gather


def kernel(x, centroids):
    x_flat = jnp.swapaxes(x, 1, -1).reshape(_N, _NF)
    idx, c_t, loss_sum = _tc_call(x_flat, centroids)
    x_q = _sc_gather_call()(c_t, idx.reshape(_NW, _CH, _CB))
    x_q = jnp.swapaxes(x_q.reshape(x.shape[0], 24, 24, _NF), 1, -1)
    loss = loss_sum[0, 0] / jnp.float32(x.size)
    return x_q, loss
